# Initial kernel scaffold; baseline (speedup 1.0000x reference)
import jax
import jax.numpy as jnp
from jax.experimental import pallas as pl


def kernel(cis, ans, ner, pos, preq, enc_hidden, params):
    raise NotImplementedError("write your pallas kernel here")


# speedup vs baseline: 2.0440x; 2.0440x over previous
import functools

import jax
import jax.numpy as jnp
from jax.experimental import pallas as pl
from jax.experimental.pallas import tpu as pltpu

B = 64
S = 256
Q = 48
HID = 256
DIN = 309
DQ = 300
TB = 8
NB = S // TB

_dot = functools.partial(jnp.dot, precision=jax.lax.Precision.HIGHEST)


def _sigmoid(x):
    return 1.0 / (1.0 + jnp.exp(-x))


def _gru_cell(gx, gh, h, m):
    z = _sigmoid(gx[:, :HID] + gh[:, :HID])
    r = _sigmoid(gx[:, HID:2 * HID] + gh[:, HID:2 * HID])
    hh = jnp.tanh(gx[:, 2 * HID:] + r * gh[:, 2 * HID:])
    h_new = z * h + (1.0 - z) * hh
    return m * h_new + (1.0 - m) * h


def _bigru_body(xf_ref, xb_ref, mf_ref, mb_ref, h0f_ref, h0b_ref,
                wf_ref, uf_ref, bif_ref, bhf_ref,
                wb_ref, ub_ref, bib_ref, bhb_ref,
                hdf_ref, hdb_ref, hf_ref, hb_ref,
                hf_scr, hb_scr):
    i = pl.program_id(0)

    @pl.when(i == 0)
    def _():
        hf_scr[...] = h0f_ref[...]
        hb_scr[...] = h0b_ref[...]

    h_f = hf_scr[...]
    h_b = hb_scr[...]
    wf = wf_ref[...]
    uf = uf_ref[...]
    wb = wb_ref[...]
    ub = ub_ref[...]
    bif = bif_ref[...]
    bhf = bhf_ref[...]
    bib = bib_ref[...]
    bhb = bhb_ref[...]

    for j in range(TB):
        xf = xf_ref[:, j, :]
        gxf = _dot(xf, wf) + bif
        ghf = _dot(h_f, uf) + bhf
        h_f = _gru_cell(gxf, ghf, h_f, mf_ref[:, j, :])
        hdf_ref[:, j, :] = h_f

        jb = TB - 1 - j
        xb = xb_ref[:, jb, :]
        gxb = _dot(xb, wb) + bib
        ghb = _dot(h_b, ub) + bhb
        h_b = _gru_cell(gxb, ghb, h_b, mb_ref[:, jb, :])
        hdb_ref[:, jb, :] = h_b

    hf_scr[...] = h_f
    hb_scr[...] = h_b
    hf_ref[...] = h_f
    hb_ref[...] = h_b


def _run_bigru(gruin, mask_f, h0f, h0b, pf, pb):
    spec_x_f = pl.BlockSpec((B, TB, DIN), lambda i: (0, i, 0))
    spec_x_b = pl.BlockSpec((B, TB, DIN), lambda i: (0, NB - 1 - i, 0))
    spec_m_f = pl.BlockSpec((B, TB, 1), lambda i: (0, i, 0))
    spec_m_b = pl.BlockSpec((B, TB, 1), lambda i: (0, NB - 1 - i, 0))
    full = lambda shape: pl.BlockSpec(shape, lambda i: (0,) * len(shape))
    out_shapes = (
        jax.ShapeDtypeStruct((B, S, HID), jnp.float32),
        jax.ShapeDtypeStruct((B, S, HID), jnp.float32),
        jax.ShapeDtypeStruct((B, HID), jnp.float32),
        jax.ShapeDtypeStruct((B, HID), jnp.float32),
    )
    out_specs = (
        pl.BlockSpec((B, TB, HID), lambda i: (0, i, 0)),
        pl.BlockSpec((B, TB, HID), lambda i: (0, NB - 1 - i, 0)),
        full((B, HID)),
        full((B, HID)),
    )
    return pl.pallas_call(
        _bigru_body,
        grid=(NB,),
        in_specs=[
            spec_x_f, spec_x_b, spec_m_f, spec_m_b,
            full((B, HID)), full((B, HID)),
            full((DIN, 3 * HID)), full((HID, 3 * HID)),
            full((1, 3 * HID)), full((1, 3 * HID)),
            full((DIN, 3 * HID)), full((HID, 3 * HID)),
            full((1, 3 * HID)), full((1, 3 * HID)),
        ],
        out_specs=out_specs,
        out_shape=out_shapes,
        scratch_shapes=[
            pltpu.VMEM((B, HID), jnp.float32),
            pltpu.VMEM((B, HID), jnp.float32),
        ],
        compiler_params=pltpu.CompilerParams(
            dimension_semantics=("arbitrary",),
        ),
    )(gruin, gruin, mask_f, mask_f, h0f, h0b,
      pf['W'], pf['U'], pf['b_i'].reshape(1, -1), pf['b_h'].reshape(1, -1),
      pb['W'], pb['U'], pb['b_i'].reshape(1, -1), pb['b_h'].reshape(1, -1))


def _qgru_final_body(xq_ref, mq_ref,
                     wqf_ref, uqf_ref, biqf_ref, bhqf_ref,
                     wqb_ref, uqb_ref, biqb_ref, bhqb_ref,
                     hf_ref, hb_ref, fw_ref, fb_ref,
                     out_ref):
    wqf = wqf_ref[...]
    uqf = uqf_ref[...]
    wqb = wqb_ref[...]
    uqb = uqb_ref[...]
    biqf = biqf_ref[...]
    bhqf = bhqf_ref[...]
    biqb = biqb_ref[...]
    bhqb = bhqb_ref[...]

    def step(j, carry):
        qf, qb = carry
        xf = xq_ref[:, pl.ds(j, 1), :].reshape(B, DQ)
        mf = mq_ref[:, pl.ds(j, 1), :].reshape(B, 1)
        gxf = _dot(xf, wqf) + biqf
        ghf = _dot(qf, uqf) + bhqf
        qf = _gru_cell(gxf, ghf, qf, mf)

        jb = Q - 1 - j
        xb = xq_ref[:, pl.ds(jb, 1), :].reshape(B, DQ)
        mb = mq_ref[:, pl.ds(jb, 1), :].reshape(B, 1)
        gxb = _dot(xb, wqb) + biqb
        ghb = _dot(qb, uqb) + bhqb
        qb = _gru_cell(gxb, ghb, qb, mb)
        return qf, qb

    zeros = jnp.zeros((B, HID), jnp.float32)
    qf, qb = jax.lax.fori_loop(0, Q, step, (zeros, zeros))

    fw = fw_ref[...]
    acc = _dot(hf_ref[...], fw[0])
    acc = acc + _dot(hb_ref[...], fw[1])
    acc = acc + _dot(qf, fw[2])
    acc = acc + _dot(qb, fw[3])
    out_ref[...] = jnp.tanh(acc + fb_ref[...])


def _run_qgru_final(preqemb, pmask_f, hf, hb, pqf, pqb, fw, fb):
    full = lambda shape: pl.BlockSpec(shape, lambda: (0,) * len(shape))
    return pl.pallas_call(
        _qgru_final_body,
        in_specs=[
            full((B, Q, DQ)), full((B, Q, 1)),
            full((DQ, 3 * HID)), full((HID, 3 * HID)),
            full((1, 3 * HID)), full((1, 3 * HID)),
            full((DQ, 3 * HID)), full((HID, 3 * HID)),
            full((1, 3 * HID)), full((1, 3 * HID)),
            full((B, HID)), full((B, HID)),
            full((4, HID, 2 * HID)), full((1, 2 * HID)),
        ],
        out_specs=full((B, 2 * HID)),
        out_shape=jax.ShapeDtypeStruct((B, 2 * HID), jnp.float32),
    )(preqemb, pmask_f,
      pqf['W'], pqf['U'], pqf['b_i'].reshape(1, -1), pqf['b_h'].reshape(1, -1),
      pqb['W'], pqb['U'], pqb['b_i'].reshape(1, -1), pqb['b_h'].reshape(1, -1),
      hf, hb, fw, fb)


def kernel(cis, ans, ner, pos, preq, enc_hidden, params):
    tokenemb = jnp.take(params['token_table'], cis, axis=0)
    preqemb = jnp.take(params['preq_table'], preq, axis=0)
    neremb = jnp.take(params['ner_table'], ner, axis=0)
    posemb = jnp.take(params['pos_table'], pos, axis=0)
    ansemb = jnp.take(params['ans_table'], ans, axis=0)

    source_mask = cis != 0
    preq_mask = preq != 0

    gruin = jnp.concatenate([tokenemb, neremb, posemb, ansemb], axis=-1)
    mask_f = source_mask.astype(jnp.float32)[:, :, None]
    pmask_f = preq_mask.astype(jnp.float32)[:, :, None]

    hd_f, hd_b, hf, hb = _run_bigru(
        gruin, mask_f, enc_hidden[0], enc_hidden[1],
        params['bigru_f'], params['bigru_b'])
    hd = jnp.concatenate([hd_f, hd_b], axis=-1)

    fw = params['final_W'].reshape(4, HID, 2 * HID)
    hD = _run_qgru_final(
        preqemb, pmask_f, hf, hb,
        params['qgru_f'], params['qgru_b'],
        fw, params['final_b'].reshape(1, -1))

    return (hd, hD, source_mask, tokenemb)
